# Initial kernel scaffold; baseline (speedup 1.0000x reference)
import jax
import jax.numpy as jnp
from jax.experimental import pallas as pl


def kernel(value, depth, position, value_table, depth_table, pos_table0, pos_table1, pos_table2, ln_gamma, ln_beta, W, b):
    raise NotImplementedError("write your pallas kernel here")


# speedup vs baseline: 1.9170x; 1.9170x over previous
import functools

import jax
import jax.numpy as jnp
from jax import lax
from jax.experimental import pallas as pl
from jax.experimental.pallas import tpu as pltpu
from jax.experimental.pallas import tpu_sc as plsc

B, T = 4, 8192
NT = B * T
D = 256
CHUNK = 8
CAT = CHUNK * D
ROWS = NT // CHUNK
OUT_D = 1024

NC, NS = 2, 16
NW = NC * NS
TPW = NT // NW
C = 64
NCHUNK = TPW // C

_LANES = 16


def _sc_embed_body(vt, dt, p0t, p1t, p2t,
                   vi, di, p0i, p1i, p2i,
                   out_hbm,
                   vi_v, di_v, p0i_v, p1i_v, p2i_v,
                   r0, r1, r2, r3, r4, sem):
    wid = lax.axis_index("s") * NC + lax.axis_index("c")
    base = wid * TPW

    def chunk_body(ci, carry):
        off = base + ci * C
        pltpu.sync_copy(vi.at[pl.ds(off, C)], vi_v)
        pltpu.sync_copy(di.at[pl.ds(off, C)], di_v)
        pltpu.sync_copy(p0i.at[pl.ds(off, C)], p0i_v)
        pltpu.sync_copy(p1i.at[pl.ds(off, C)], p1i_v)
        pltpu.sync_copy(p2i.at[pl.ds(off, C)], p2i_v)
        c0 = pltpu.async_copy(vt.at[vi_v], r0, sem)
        c1 = pltpu.async_copy(dt.at[di_v], r1, sem)
        c2 = pltpu.async_copy(p0t.at[p0i_v], r2, sem)
        c3 = pltpu.async_copy(p1t.at[p1i_v], r3, sem)
        c4 = pltpu.async_copy(p2t.at[p2i_v], r4, sem)
        c0.wait()
        c1.wait()
        c2.wait()
        c3.wait()
        c4.wait()

        def sum_body(i, carry2):
            for j in range(D // _LANES):
                s = pl.ds(j * _LANES, _LANES)
                r0[i, s] = r0[i, s] + r1[i, s] + r2[i, s] + r3[i, s] + r4[i, s]
            return carry2

        lax.fori_loop(0, C, sum_body, 0, unroll=False)
        pltpu.sync_copy(r0, out_hbm.at[pl.ds(off, C)])
        return carry

    lax.fori_loop(0, NCHUNK, chunk_body, 0, unroll=False)


@functools.partial(
    pl.kernel,
    out_type=jax.ShapeDtypeStruct((NT, D), jnp.float32),
    mesh=plsc.VectorSubcoreMesh(core_axis_name="c", subcore_axis_name="s",
                                num_cores=NC, num_subcores=NS),
    scratch_types=[
        pltpu.VMEM((C,), jnp.int32),
        pltpu.VMEM((C,), jnp.int32),
        pltpu.VMEM((C,), jnp.int32),
        pltpu.VMEM((C,), jnp.int32),
        pltpu.VMEM((C,), jnp.int32),
        pltpu.VMEM((C, D), jnp.float32),
        pltpu.VMEM((C, D), jnp.float32),
        pltpu.VMEM((C, D), jnp.float32),
        pltpu.VMEM((C, D), jnp.float32),
        pltpu.VMEM((C, D), jnp.float32),
        pltpu.SemaphoreType.DMA,
    ],
)
def _sc_embed(*refs):
    _sc_embed_body(*refs)


def _tc_mlp_body(x_ref, g_ref, bt_ref, w_ref, b_ref, o_ref):
    x = x_ref[...]
    mu = jnp.mean(x, axis=-1, keepdims=True)
    xc = x - mu
    var = jnp.mean(xc * xc, axis=-1, keepdims=True)
    xn = xc * lax.rsqrt(var + 1e-5)
    xn = xn * g_ref[...] + bt_ref[...]
    y = jnp.dot(xn, w_ref[...], preferred_element_type=jnp.float32) + b_ref[...]
    o_ref[...] = 0.5 * y * (1.0 + lax.erf(y * 0.7071067811865476))


def _tc_mlp(x, g, bt, w, b):
    bm = 512
    return pl.pallas_call(
        _tc_mlp_body,
        grid=(ROWS // bm,),
        in_specs=[
            pl.BlockSpec((bm, CAT), lambda i: (i, 0)),
            pl.BlockSpec((1, CAT), lambda i: (0, 0)),
            pl.BlockSpec((1, CAT), lambda i: (0, 0)),
            pl.BlockSpec((CAT, OUT_D), lambda i: (0, 0)),
            pl.BlockSpec((1, OUT_D), lambda i: (0, 0)),
        ],
        out_specs=pl.BlockSpec((bm, OUT_D), lambda i: (i, 0)),
        out_shape=jax.ShapeDtypeStruct((ROWS, OUT_D), jnp.float32),
    )(x, g, bt, w, b)


def kernel(value, depth, position, value_table, depth_table,
           pos_table0, pos_table1, pos_table2, ln_gamma, ln_beta, W, b):
    vi = value.reshape(-1).astype(jnp.int32)
    di = depth.reshape(-1).astype(jnp.int32)
    pos = position.astype(jnp.int32)
    p0i = pos[:, :, 0].reshape(-1)
    p1i = pos[:, :, 1].reshape(-1)
    p2i = pos[:, :, 2].reshape(-1)

    x = _sc_embed(value_table, depth_table, pos_table0, pos_table1, pos_table2,
                  vi, di, p0i, p1i, p2i)
    x = x.reshape(ROWS, CAT)
    out = _tc_mlp(x, ln_gamma.reshape(1, CAT), ln_beta.reshape(1, CAT),
                  W, b.reshape(1, OUT_D))
    return out.reshape(B, ROWS // B, OUT_D)


# speedup vs baseline: 1.9413x; 1.0127x over previous
import functools

import jax
import jax.numpy as jnp
from jax import lax
from jax.experimental import pallas as pl
from jax.experimental.pallas import tpu as pltpu
from jax.experimental.pallas import tpu_sc as plsc

B, T = 4, 8192
NT = B * T
D = 256
CHUNK = 8
CAT = CHUNK * D
ROWS = NT // CHUNK
OUT_D = 1024

NC, NS = 2, 16
NW = NC * NS
TPW = NT // NW
C = 32
NCHUNK = TPW // C
NBUF = 2

_LANES = 16


def _sc_embed_body(vt, dt, p0t, p1t, p2t,
                   vi, di, p0i, p1i, p2i, out_hbm, *sc):
    tables = (vt, dt, p0t, p1t, p2t)
    idxsrc = (vi, di, p0i, p1i, p2i)
    idxb = (sc[0:5], sc[5:10])
    rows = (sc[10:15], sc[15:20])
    semi = (sc[20], sc[21])
    semg = (sc[22], sc[23])
    semo = (sc[24], sc[25])

    wid = lax.axis_index("s") * NC + lax.axis_index("c")
    base = wid * TPW

    def fire_idx(ci, p):
        off = base + ci * C
        for k in range(5):
            pltpu.async_copy(idxsrc[k].at[pl.ds(off, C)], idxb[p][k], semi[p])

    def wait_idx(ci, p):
        off = base + ci * C
        for k in range(5):
            pltpu.make_async_copy(idxsrc[k].at[pl.ds(off, C)], idxb[p][k],
                                  semi[p]).wait()

    def fire_gathers(p):
        for k in range(5):
            pltpu.async_copy(tables[k].at[idxb[p][k]], rows[p][k], semg[p])

    def drain_gathers(p):
        for k in range(5):
            pltpu.make_async_copy(tables[k].at[idxb[p][k]], rows[p][k],
                                  semg[p]).wait()

    def wait_out(p):
        pltpu.make_async_copy(rows[p][0], out_hbm.at[pl.ds(base, C)],
                              semo[p]).wait()

    def sum_and_out(ci, p):
        r0, r1, r2, r3, r4 = rows[p]

        def sum_body(i, carry):
            for j in range(D // _LANES):
                s = pl.ds(j * _LANES, _LANES)
                r0[i, s] = r0[i, s] + r1[i, s] + r2[i, s] + r3[i, s] + r4[i, s]
            return carry

        lax.fori_loop(0, C, sum_body, 0, unroll=False)
        off = base + ci * C
        pltpu.async_copy(r0, out_hbm.at[pl.ds(off, C)], semo[p])

    fire_idx(0, 0)
    wait_idx(0, 0)
    fire_gathers(0)
    fire_idx(1, 1)

    @pl.loop(0, NCHUNK, step=NBUF)
    def _outer(g):
        for b in range(NBUF):
            ci = g + b
            nci = ci + 1
            drain_gathers(b)

            @pl.when(ci + 2 < NCHUNK)
            def _():
                fire_idx(ci + 2, b)

            @pl.when(nci < NCHUNK)
            def _():
                wait_idx(nci, 1 - b)

                @pl.when(nci >= NBUF)
                def _():
                    wait_out(1 - b)
                fire_gathers(1 - b)

            sum_and_out(ci, b)

    wait_out(0)
    wait_out(1)


@functools.partial(
    pl.kernel,
    out_type=jax.ShapeDtypeStruct((NT, D), jnp.float32),
    mesh=plsc.VectorSubcoreMesh(core_axis_name="c", subcore_axis_name="s",
                                num_cores=NC, num_subcores=NS),
    scratch_types=(
        [pltpu.VMEM((C,), jnp.int32)] * (5 * NBUF)
        + [pltpu.VMEM((C, D), jnp.float32)] * (5 * NBUF)
        + [pltpu.SemaphoreType.DMA] * (3 * NBUF)
    ),
)
def _sc_embed(*refs):
    _sc_embed_body(*refs)


def _tc_mlp_body(x_ref, g_ref, bt_ref, w_ref, b_ref, o_ref):
    x = x_ref[...]
    mu = jnp.mean(x, axis=-1, keepdims=True)
    xc = x - mu
    var = jnp.mean(xc * xc, axis=-1, keepdims=True)
    xn = xc * lax.rsqrt(var + 1e-5)
    xn = xn * g_ref[...] + bt_ref[...]
    y = jnp.dot(xn, w_ref[...], preferred_element_type=jnp.float32) + b_ref[...]
    o_ref[...] = 0.5 * y * (1.0 + lax.erf(y * 0.7071067811865476))


def _tc_mlp(x, g, bt, w, b):
    bm = 512
    return pl.pallas_call(
        _tc_mlp_body,
        grid=(ROWS // bm,),
        in_specs=[
            pl.BlockSpec((bm, CAT), lambda i: (i, 0)),
            pl.BlockSpec((1, CAT), lambda i: (0, 0)),
            pl.BlockSpec((1, CAT), lambda i: (0, 0)),
            pl.BlockSpec((CAT, OUT_D), lambda i: (0, 0)),
            pl.BlockSpec((1, OUT_D), lambda i: (0, 0)),
        ],
        out_specs=pl.BlockSpec((bm, OUT_D), lambda i: (i, 0)),
        out_shape=jax.ShapeDtypeStruct((ROWS, OUT_D), jnp.float32),
    )(x, g, bt, w, b)


def kernel(value, depth, position, value_table, depth_table,
           pos_table0, pos_table1, pos_table2, ln_gamma, ln_beta, W, b):
    vi = value.reshape(-1).astype(jnp.int32)
    di = depth.reshape(-1).astype(jnp.int32)
    pos = position.astype(jnp.int32)
    p0i = pos[:, :, 0].reshape(-1)
    p1i = pos[:, :, 1].reshape(-1)
    p2i = pos[:, :, 2].reshape(-1)

    x = _sc_embed(value_table, depth_table, pos_table0, pos_table1, pos_table2,
                  vi, di, p0i, p1i, p2i)
    x = x.reshape(ROWS, CAT)
    out = _tc_mlp(x, ln_gamma.reshape(1, CAT), ln_beta.reshape(1, CAT),
                  W, b.reshape(1, OUT_D))
    return out.reshape(B, ROWS // B, OUT_D)


# speedup vs baseline: 1.9648x; 1.0121x over previous
import functools

import jax
import jax.numpy as jnp
from jax import lax
from jax.experimental import pallas as pl
from jax.experimental.pallas import tpu as pltpu
from jax.experimental.pallas import tpu_sc as plsc

B, T = 4, 8192
NT = B * T
D = 256
CHUNK = 8
CAT = CHUNK * D
ROWS = NT // CHUNK
OUT_D = 1024

NC, NS = 2, 16
NW = NC * NS
TPW = NT // NW
C = 32
NCHUNK = TPW // C
NBUF = 2

_LANES = 16


def _sc_embed_body(vt, dt, p0t, p1t, p2t,
                   vi, di, p0i, p1i, p2i, out_hbm, *sc):
    tables = (vt, dt, p0t, p1t, p2t)
    idxsrc = (vi, di, p0i, p1i, p2i)
    idxb = (sc[0:5], sc[5:10])
    rows = (sc[10:15], sc[15:20])
    rout = (sc[20], sc[21])
    semi = (sc[22], sc[23])
    semg = (sc[24], sc[25])
    semo = (sc[26], sc[27])

    wid = lax.axis_index("s") * NC + lax.axis_index("c")
    base = wid * TPW

    def fire_idx(ci, p):
        off = base + ci * C
        for k in range(5):
            pltpu.async_copy(idxsrc[k].at[pl.ds(off, C)], idxb[p][k], semi[p])

    def wait_idx(ci, p):
        off = base + ci * C
        for k in range(5):
            pltpu.make_async_copy(idxsrc[k].at[pl.ds(off, C)], idxb[p][k],
                                  semi[p]).wait()

    def fire_gathers(p):
        for k in range(5):
            pltpu.async_copy(tables[k].at[idxb[p][k]], rows[p][k], semg[p])

    def drain_gathers(p):
        for k in range(5):
            pltpu.make_async_copy(tables[k].at[idxb[p][k]], rows[p][k],
                                  semg[p]).wait()

    def wait_out(p):
        pltpu.make_async_copy(rout[p], out_hbm.at[pl.ds(base, C)],
                              semo[p]).wait()

    def sum_and_out(ci, p):
        r0, r1, r2, r3, r4 = rows[p]
        ro = rout[p]

        def sum_body(i, carry):
            for j in range(D // _LANES):
                s = pl.ds(j * _LANES, _LANES)
                ro[i, s] = ((r0[i, s] + r1[i, s]) + (r2[i, s] + r3[i, s])
                            + r4[i, s])
            return carry

        lax.fori_loop(0, C, sum_body, 0, unroll=False)
        off = base + ci * C
        pltpu.async_copy(ro, out_hbm.at[pl.ds(off, C)], semo[p])

    fire_idx(0, 0)
    wait_idx(0, 0)
    fire_gathers(0)
    fire_idx(1, 1)

    @pl.loop(0, NCHUNK, step=NBUF)
    def _outer(g):
        for b in range(NBUF):
            ci = g + b
            nci = ci + 1
            drain_gathers(b)

            @pl.when(ci + 2 < NCHUNK)
            def _():
                fire_idx(ci + 2, b)

            @pl.when(nci < NCHUNK)
            def _():
                wait_idx(nci, 1 - b)
                fire_gathers(1 - b)

            @pl.when(ci >= NBUF)
            def _():
                wait_out(b)

            sum_and_out(ci, b)

    wait_out(0)
    wait_out(1)


@functools.partial(
    pl.kernel,
    out_type=jax.ShapeDtypeStruct((NT, D), jnp.float32),
    mesh=plsc.VectorSubcoreMesh(core_axis_name="c", subcore_axis_name="s",
                                num_cores=NC, num_subcores=NS),
    scratch_types=(
        [pltpu.VMEM((C,), jnp.int32)] * (5 * NBUF)
        + [pltpu.VMEM((C, D), jnp.float32)] * (6 * NBUF)
        + [pltpu.SemaphoreType.DMA] * (3 * NBUF)
    ),
)
def _sc_embed(*refs):
    _sc_embed_body(*refs)


def _tc_mlp_body(x_ref, g_ref, bt_ref, w_ref, b_ref, o_ref):
    x = x_ref[...]
    mu = jnp.mean(x, axis=-1, keepdims=True)
    xc = x - mu
    var = jnp.mean(xc * xc, axis=-1, keepdims=True)
    xn = xc * lax.rsqrt(var + 1e-5)
    xn = xn * g_ref[...] + bt_ref[...]
    y = jnp.dot(xn, w_ref[...], preferred_element_type=jnp.float32) + b_ref[...]
    o_ref[...] = 0.5 * y * (1.0 + lax.erf(y * 0.7071067811865476))


def _tc_mlp(x, g, bt, w, b):
    bm = 512
    return pl.pallas_call(
        _tc_mlp_body,
        grid=(ROWS // bm,),
        in_specs=[
            pl.BlockSpec((bm, CAT), lambda i: (i, 0)),
            pl.BlockSpec((1, CAT), lambda i: (0, 0)),
            pl.BlockSpec((1, CAT), lambda i: (0, 0)),
            pl.BlockSpec((CAT, OUT_D), lambda i: (0, 0)),
            pl.BlockSpec((1, OUT_D), lambda i: (0, 0)),
        ],
        out_specs=pl.BlockSpec((bm, OUT_D), lambda i: (i, 0)),
        out_shape=jax.ShapeDtypeStruct((ROWS, OUT_D), jnp.float32),
    )(x, g, bt, w, b)


def kernel(value, depth, position, value_table, depth_table,
           pos_table0, pos_table1, pos_table2, ln_gamma, ln_beta, W, b):
    vi = value.reshape(-1).astype(jnp.int32)
    di = depth.reshape(-1).astype(jnp.int32)
    pos = position.astype(jnp.int32)
    p0i = pos[:, :, 0].reshape(-1)
    p1i = pos[:, :, 1].reshape(-1)
    p2i = pos[:, :, 2].reshape(-1)

    x = _sc_embed(value_table, depth_table, pos_table0, pos_table1, pos_table2,
                  vi, di, p0i, p1i, p2i)
    x = x.reshape(ROWS, CAT)
    out = _tc_mlp(x, ln_gamma.reshape(1, CAT), ln_beta.reshape(1, CAT),
                  W, b.reshape(1, OUT_D))
    return out.reshape(B, ROWS // B, OUT_D)


# speedup vs baseline: 2.5072x; 1.2761x over previous
import functools

import jax
import jax.numpy as jnp
from jax import lax
from jax.experimental import pallas as pl
from jax.experimental.pallas import tpu as pltpu
from jax.experimental.pallas import tpu_sc as plsc

B, T = 4, 8192
NT = B * T
D = 256
CHUNK = 8
CAT = CHUNK * D
ROWS = NT // CHUNK
OUT_D = 1024

NC, NS = 2, 16
NW = NC * NS
TPW = NT // NW
C = 16
NCHUNK = TPW // C
NBUF = 2

_LANES = 16


def _sc_embed_body(vt, dt, p0t, p1t, p2t,
                   vi, di, p0i, p1i, p2i, out_hbm, *sc):
    vib = (sc[0], sc[1])
    p2b = (sc[2], sc[3])
    dib = (sc[4], sc[5])
    p0b = (sc[6], sc[7])
    p1b = (sc[8], sc[9])
    rv = (sc[10], sc[11])
    rp2 = (sc[12], sc[13])
    dtv, p0v, p1v = sc[14:17]
    semi = (sc[17], sc[18])
    semg = (sc[19], sc[20])
    semo = (sc[21], sc[22])

    wid = lax.axis_index("s") * NC + lax.axis_index("c")
    base = wid * TPW

    pltpu.sync_copy(dt, dtv)
    pltpu.sync_copy(p0t, p0v)
    pltpu.sync_copy(p1t, p1v)

    def fire_idx(ci, p):
        off = base + ci * C
        pltpu.async_copy(vi.at[pl.ds(off, C)], vib[p], semi[p])
        pltpu.async_copy(p2i.at[pl.ds(off, C)], p2b[p], semi[p])
        pltpu.async_copy(di.at[pl.ds(off, C)], dib[p], semi[p])
        pltpu.async_copy(p0i.at[pl.ds(off, C)], p0b[p], semi[p])
        pltpu.async_copy(p1i.at[pl.ds(off, C)], p1b[p], semi[p])

    def wait_idx(ci, p):
        off = base + ci * C
        pltpu.make_async_copy(vi.at[pl.ds(off, C)], vib[p], semi[p]).wait()
        pltpu.make_async_copy(p2i.at[pl.ds(off, C)], p2b[p], semi[p]).wait()
        pltpu.make_async_copy(di.at[pl.ds(off, C)], dib[p], semi[p]).wait()
        pltpu.make_async_copy(p0i.at[pl.ds(off, C)], p0b[p], semi[p]).wait()
        pltpu.make_async_copy(p1i.at[pl.ds(off, C)], p1b[p], semi[p]).wait()

    colc = [jnp.full((_LANES,), j * _LANES, jnp.int32)
            for j in range(D // _LANES)]

    def fire_gathers(p):
        pltpu.async_copy(vt.at[vib[p]], rv[p], semg[p])
        pltpu.async_copy(p2t.at[p2b[p]], rp2[p], semg[p])

    def drain_gathers(p):
        pltpu.make_async_copy(vt.at[vib[p]], rv[p], semg[p]).wait()
        pltpu.make_async_copy(p2t.at[p2b[p]], rp2[p], semg[p]).wait()

    def wait_out(p):
        pltpu.make_async_copy(rv[p], out_hbm.at[pl.ds(base, C)],
                              semo[p]).wait()

    def sum_and_out(ci, p):
        r = rv[p]
        r2 = rp2[p]
        idxd, idxp0, idxp1 = dib[p], p0b[p], p1b[p]

        def sum_body(i, carry):
            dvec = idxd[i, :]
            p0vec = idxp0[i, :]
            p1vec = idxp1[i, :]
            for j in range(D // _LANES):
                s = pl.ds(j * _LANES, _LANES)
                acc = r[i, s] + r2[i, s]
                acc = acc + plsc.load_gather(dtv, [dvec + colc[j]])
                acc = acc + plsc.load_gather(p0v, [p0vec + colc[j]])
                acc = acc + plsc.load_gather(p1v, [p1vec + colc[j]])
                r[i, s] = acc
            return carry

        lax.fori_loop(0, C, sum_body, 0, unroll=False)
        off = base + ci * C
        pltpu.async_copy(r, out_hbm.at[pl.ds(off, C)], semo[p])

    fire_idx(0, 0)
    wait_idx(0, 0)
    fire_gathers(0)
    fire_idx(1, 1)

    @pl.loop(0, NCHUNK, step=NBUF)
    def _outer(g):
        for b in range(NBUF):
            ci = g + b
            nci = ci + 1
            drain_gathers(b)

            @pl.when(nci < NCHUNK)
            def _():
                wait_idx(nci, 1 - b)

                @pl.when(nci >= NBUF)
                def _():
                    wait_out(1 - b)
                fire_gathers(1 - b)

            sum_and_out(ci, b)

            @pl.when(ci + 2 < NCHUNK)
            def _():
                fire_idx(ci + 2, b)

    wait_out(0)
    wait_out(1)


@functools.partial(
    pl.kernel,
    out_type=jax.ShapeDtypeStruct((NT, D), jnp.float32),
    mesh=plsc.VectorSubcoreMesh(core_axis_name="c", subcore_axis_name="s",
                                num_cores=NC, num_subcores=NS),
    compiler_params=pltpu.CompilerParams(needs_layout_passes=False),
    scratch_types=(
        [pltpu.VMEM((C,), jnp.int32)] * 4
        + [pltpu.VMEM((C, _LANES), jnp.int32)] * 6
        + [pltpu.VMEM((C, D), jnp.float32)] * 4
        + [pltpu.VMEM((7 * D,), jnp.float32)]
        + [pltpu.VMEM((128 * D,), jnp.float32)] * 2
        + [pltpu.SemaphoreType.DMA] * (3 * NBUF)
    ),
)
def _sc_embed(*refs):
    _sc_embed_body(*refs)


def _tc_mlp_body(x_ref, g_ref, bt_ref, w_ref, b_ref, o_ref):
    x = x_ref[...]
    mu = jnp.mean(x, axis=-1, keepdims=True)
    xc = x - mu
    var = jnp.mean(xc * xc, axis=-1, keepdims=True)
    xn = xc * lax.rsqrt(var + 1e-5)
    xn = xn * g_ref[...] + bt_ref[...]
    y = jnp.dot(xn, w_ref[...], preferred_element_type=jnp.float32) + b_ref[...]
    o_ref[...] = 0.5 * y * (1.0 + lax.erf(y * 0.7071067811865476))


def _tc_mlp(x, g, bt, w, b):
    bm = 512
    return pl.pallas_call(
        _tc_mlp_body,
        grid=(ROWS // bm,),
        in_specs=[
            pl.BlockSpec((bm, CAT), lambda i: (i, 0)),
            pl.BlockSpec((1, CAT), lambda i: (0, 0)),
            pl.BlockSpec((1, CAT), lambda i: (0, 0)),
            pl.BlockSpec((CAT, OUT_D), lambda i: (0, 0)),
            pl.BlockSpec((1, OUT_D), lambda i: (0, 0)),
        ],
        out_specs=pl.BlockSpec((bm, OUT_D), lambda i: (i, 0)),
        out_shape=jax.ShapeDtypeStruct((ROWS, OUT_D), jnp.float32),
    )(x, g, bt, w, b)


def kernel(value, depth, position, value_table, depth_table,
           pos_table0, pos_table1, pos_table2, ln_gamma, ln_beta, W, b):
    vi = value.reshape(-1).astype(jnp.int32)
    di = depth.reshape(-1).astype(jnp.int32)
    pos = position.astype(jnp.int32)
    p0i = pos[:, :, 0].reshape(-1)
    p1i = pos[:, :, 1].reshape(-1)
    p2i = pos[:, :, 2].reshape(-1)

    lane = jnp.arange(_LANES, dtype=jnp.int32)
    x = _sc_embed(value_table, depth_table.reshape(-1),
                  pos_table0.reshape(-1), pos_table1.reshape(-1),
                  pos_table2, vi,
                  (di * D)[:, None] + lane, (p0i * D)[:, None] + lane,
                  (p1i * D)[:, None] + lane, p2i)
    x = x.reshape(ROWS, CAT)
    out = _tc_mlp(x, ln_gamma.reshape(1, CAT), ln_beta.reshape(1, CAT),
                  W, b.reshape(1, OUT_D))
    return out.reshape(B, ROWS // B, OUT_D)
